# Initial kernel scaffold; baseline (speedup 1.0000x reference)
import functools

import jax
import jax.numpy as jnp
from jax import lax
from jax.experimental import pallas as pl
from jax.experimental.pallas import tpu as pltpu
from jax.experimental.pallas import tpu_sc as plsc


@functools.lru_cache(maxsize=None)
def _make_gather(n: int, v: int, d: int, chunk: int):
    info = plsc.get_sparse_core_info()
    nw = info.num_cores * info.num_subcores
    assert n % nw == 0
    per_w = n // nw
    assert per_w % chunk == 0
    n_chunks = per_w // chunk
    mesh = plsc.VectorSubcoreMesh(core_axis_name="c", subcore_axis_name="s")

    @functools.partial(
        pl.kernel,
        mesh=mesh,
        out_type=jax.ShapeDtypeStruct((n, d), jnp.float32),
        scratch_types=[
            pltpu.VMEM((chunk,), jnp.int32),
            pltpu.VMEM((chunk, d), jnp.float32),
            pltpu.SemaphoreType.DMA,
        ],
    )
    def gather_kernel(idx_hbm, table_hbm, out_hbm, idx_v, rows_v, sem):
        wid = lax.axis_index("s") * info.num_cores + lax.axis_index("c")
        base = wid * per_w

        def body(j, carry):
            off = base + j * chunk
            pltpu.sync_copy(idx_hbm.at[pl.ds(off, chunk)], idx_v)
            pltpu.async_copy(table_hbm.at[idx_v], rows_v, sem).wait()
            pltpu.sync_copy(rows_v, out_hbm.at[pl.ds(off, chunk)])
            return carry

        lax.fori_loop(0, n_chunks, body, 0)

    return gather_kernel


def kernel(t_index, emb):
    b, t = t_index.shape
    v, d = emb.shape
    flat = t_index.reshape(-1)
    out = _make_gather(flat.shape[0], v, d, 1024)(flat, emb)
    return out.reshape(b, t, d)


# speedup vs baseline: 1.4614x; 1.4614x over previous
import functools

import jax
import jax.numpy as jnp
from jax import lax
from jax.experimental import pallas as pl
from jax.experimental.pallas import tpu as pltpu
from jax.experimental.pallas import tpu_sc as plsc


@functools.lru_cache(maxsize=None)
def _make_gather(n: int, v: int, d: int, chunk: int):
    info = plsc.get_sparse_core_info()
    nw = info.num_cores * info.num_subcores
    assert n % nw == 0
    per_w = n // nw
    assert per_w % chunk == 0
    n_chunks = per_w // chunk
    mesh = plsc.VectorSubcoreMesh(core_axis_name="c", subcore_axis_name="s")

    @functools.partial(
        pl.kernel,
        mesh=mesh,
        out_type=jax.ShapeDtypeStruct((n, d), jnp.float32),
        scratch_types=[
            pltpu.VMEM((chunk,), jnp.int32),
            pltpu.VMEM((chunk, d), jnp.float32),
            pltpu.SemaphoreType.DMA,
        ],
        compiler_params=pltpu.CompilerParams(use_tc_tiling_on_sc=False),
    )
    def gather_kernel(idx_hbm, table_hbm, out_hbm, idx_v, rows_v, sem):
        wid = lax.axis_index("s") * info.num_cores + lax.axis_index("c")
        base = wid * per_w

        def body(j, carry):
            off = base + j * chunk
            pltpu.sync_copy(idx_hbm.at[pl.ds(off, chunk)], idx_v)
            pltpu.async_copy(table_hbm.at[idx_v], rows_v, sem).wait()
            pltpu.sync_copy(rows_v, out_hbm.at[pl.ds(off, chunk)])
            return carry

        lax.fori_loop(0, n_chunks, body, 0)

    return gather_kernel


def kernel(t_index, emb):
    b, t = t_index.shape
    v, d = emb.shape
    flat = t_index.reshape(-1)
    out = _make_gather(flat.shape[0], v, d, 1024)(flat, emb)
    return out.reshape(b, t, d)


# speedup vs baseline: 1.5044x; 1.0295x over previous
import functools

import jax
import jax.numpy as jnp
from jax import lax
from jax.experimental import pallas as pl
from jax.experimental.pallas import tpu as pltpu
from jax.experimental.pallas import tpu_sc as plsc


@functools.lru_cache(maxsize=None)
def _make_gather(n: int, d: int, chunk: int, nbuf: int):
    info = plsc.get_sparse_core_info()
    nw = info.num_cores * info.num_subcores
    assert n % nw == 0
    per_w = n // nw
    assert per_w % chunk == 0
    k_chunks = per_w // chunk
    assert k_chunks % nbuf == 0
    groups = k_chunks // nbuf
    skew = nbuf - 1
    mesh = plsc.VectorSubcoreMesh(core_axis_name="c", subcore_axis_name="s")

    scratch = (
        [pltpu.VMEM((chunk,), jnp.int32) for _ in range(nbuf)]
        + [pltpu.VMEM((chunk, d), jnp.float32) for _ in range(nbuf)]
        + [pltpu.SemaphoreType.DMA] * (3 * nbuf)
    )

    @functools.partial(
        pl.kernel,
        mesh=mesh,
        out_type=jax.ShapeDtypeStruct((n, d), jnp.float32),
        scratch_types=scratch,
        compiler_params=pltpu.CompilerParams(use_tc_tiling_on_sc=False),
    )
    def gather_kernel(idx_hbm, table_hbm, out_hbm, *scr):
        idx_v = scr[:nbuf]
        rows_v = scr[nbuf : 2 * nbuf]
        sem_i = scr[2 * nbuf : 3 * nbuf]
        sem_g = scr[3 * nbuf : 4 * nbuf]
        sem_s = scr[4 * nbuf : 5 * nbuf]
        wid = lax.axis_index("s") * info.num_cores + lax.axis_index("c")
        base = wid * per_w

        def start_idx(j, b):
            pltpu.async_copy(
                idx_hbm.at[pl.ds(base + j * chunk, chunk)], idx_v[b], sem_i[b]
            )

        def wait_idx(b):
            pltpu.make_async_copy(
                idx_hbm.at[pl.ds(base, chunk)], idx_v[b], sem_i[b]
            ).wait()

        def start_gather(b):
            pltpu.async_copy(table_hbm.at[idx_v[b]], rows_v[b], sem_g[b])

        def wait_gather(b):
            pltpu.make_async_copy(
                table_hbm.at[idx_v[b]], rows_v[b], sem_g[b]
            ).wait()

        def start_store(j, b):
            pltpu.async_copy(
                rows_v[b], out_hbm.at[pl.ds(base + j * chunk, chunk)], sem_s[b]
            )

        def wait_store(b):
            pltpu.make_async_copy(
                rows_v[b], out_hbm.at[pl.ds(base, chunk)], sem_s[b]
            ).wait()

        start_idx(0, 0)

        def group(g, carry):
            for b in range(nbuf):
                j = g * nbuf + b

                @pl.when(g >= 1)
                def _():
                    wait_store(b)

                wait_idx(b)
                start_gather(b)

                bs = (b + 1) % nbuf
                if b == nbuf - 1:
                    wait_gather(bs)
                    start_store(j - skew, bs)
                else:

                    @pl.when(g >= 1)
                    def _():
                        wait_gather(bs)
                        start_store(j - skew, bs)

                bn = (b + 1) % nbuf
                if b < nbuf - 1:
                    start_idx(j + 1, bn)
                else:

                    @pl.when(g < groups - 1)
                    def _():
                        start_idx(j + 1, bn)

            return carry

        lax.fori_loop(0, groups, group, 0)

        for t in range(k_chunks - skew, k_chunks):
            b = t % nbuf
            wait_gather(b)
            start_store(t, b)
        for b in range(nbuf):
            wait_store(b)

    return gather_kernel


def kernel(t_index, emb):
    b, t = t_index.shape
    v, d = emb.shape
    flat = t_index.reshape(-1)
    out = _make_gather(flat.shape[0], d, 512, 5)(flat, emb)
    return out.reshape(b, t, d)


# speedup vs baseline: 2.2949x; 1.5254x over previous
import functools

import jax
import jax.numpy as jnp
from jax import lax
from jax.experimental import pallas as pl
from jax.experimental.pallas import tpu as pltpu
from jax.experimental.pallas import tpu_sc as plsc


@functools.lru_cache(maxsize=None)
def _make_gather(bsz: int, tsz: int, d: int):
    n = bsz * tsz
    info = plsc.get_sparse_core_info()
    nw = info.num_cores * info.num_subcores
    nb_total = bsz // 128
    pairs = tsz * nb_total
    assert pairs % nw == 0
    per_w = pairs // nw
    pg = 4
    assert per_w % (2 * pg) == 0
    k_groups = per_w // pg
    chunk = pg * 128
    njb = d // 8
    mesh = plsc.VectorSubcoreMesh(core_axis_name="c", subcore_axis_name="s")

    scratch = (
        [pltpu.VMEM((per_w * 128,), jnp.int32)]
        + [pltpu.VMEM((chunk, d), jnp.float32) for _ in range(2)]
        + [pltpu.VMEM((pg * d, 129), jnp.float32) for _ in range(2)]
        + [pltpu.SemaphoreType.DMA] * 5
    )

    @functools.partial(
        pl.kernel,
        mesh=mesh,
        out_type=jax.ShapeDtypeStruct((tsz, njb, nb_total, 8, 128), jnp.float32),
        scratch_types=scratch,
        compiler_params=pltpu.CompilerParams(
            use_tc_tiling_on_sc=False, needs_layout_passes=False
        ),
    )
    def gather_kernel(idx_hbm, table_hbm, out_hbm, idx_v, r0, r1, o0, o1,
                      sem_i, sg0, sg1, ss0, ss1):
        rows_v = (r0, r1)
        out_v = (o0, o1)
        sem_g = (sg0, sg1)
        sem_s = (ss0, ss1)
        wid = lax.axis_index("s") * info.num_cores + lax.axis_index("c")
        base_pair = wid * per_w

        pltpu.sync_copy(idx_hbm.at[pl.ds(base_pair * 128, per_w * 128)], idx_v)

        iota = lax.iota(jnp.int32, 16)

        def start_gather(q, b):
            pltpu.async_copy(
                table_hbm.at[idx_v.at[pl.ds(q * chunk, chunk)]],
                rows_v[b], sem_g[b],
            )

        def wait_gather(b):
            pltpu.make_async_copy(
                table_hbm.at[idx_v.at[pl.ds(0, chunk)]], rows_v[b], sem_g[b]
            ).wait()

        def start_store(q, b):
            for p in range(pg):
                pair = base_pair + q * pg + p
                t = pair // nb_total
                nb = lax.rem(pair, nb_total)
                for jb in range(njb):
                    pltpu.async_copy(
                        out_v[b].at[pl.ds((p * njb + jb) * 8, 8), pl.ds(0, 128)],
                        out_hbm.at[t, jb, nb],
                        sem_s[b],
                    )

        def wait_store(b):
            for p in range(pg):
                for jb in range(njb):
                    pltpu.make_async_copy(
                        out_v[b].at[pl.ds((p * njb + jb) * 8, 8), pl.ds(0, 128)],
                        out_hbm.at[0, jb, 0],
                        sem_s[b],
                    ).wait()

        def transpose(b):
            for p in range(pg):
                row_lo = iota + (p * d)
                row_hi = row_lo + 16

                def body(nl, carry):
                    r = p * 128 + nl
                    nl_vec = jnp.zeros((16,), jnp.int32) + nl
                    lo = rows_v[b][r, pl.ds(0, 16)]
                    hi = rows_v[b][r, pl.ds(16, 16)]
                    plsc.store_scatter(out_v[b], [row_lo, nl_vec], lo)
                    plsc.store_scatter(out_v[b], [row_hi, nl_vec], hi)
                    return carry

                lax.fori_loop(0, 128, body, 0, unroll=4)

        start_gather(0, 0)

        def group(g, carry):
            for b in range(2):
                q = g * 2 + b

                @pl.when(q < k_groups - 1)
                def _():
                    start_gather(q + 1, 1 - b)

                wait_gather(b)

                @pl.when(g >= 1)
                def _():
                    wait_store(b)

                transpose(b)
                start_store(q, b)
            return carry

        lax.fori_loop(0, k_groups // 2, group, 0)
        wait_store(0)
        wait_store(1)

    return gather_kernel


def kernel(t_index, emb):
    b, t = t_index.shape
    v, d = emb.shape
    idx_t_flat = t_index.T.reshape(-1)
    ot5 = _make_gather(b, t, d)(idx_t_flat, emb)
    return ot5.transpose(2, 4, 0, 1, 3).reshape(b, t, d)
